# baseline (speedup 1.0000x reference)
import functools
import math

import jax
import jax.numpy as jnp
from jax.experimental import pallas as pl
from jax.experimental.pallas import tpu as pltpu

DIM = 128
KV_LEN = 8192
BB = 2
NSPLIT = 4
CHUNK = KV_LEN // NSPLIT


def _attn_kernel(q_ref, *refs):
    k_refs = refs[:NSPLIT]
    v_refs = refs[NSPLIT:2 * NSPLIT]
    ctx_ref, attn_ref = refs[2 * NSPLIT], refs[2 * NSPLIT + 1]
    scale = 1.0 / math.sqrt(DIM)
    for i in range(BB):
        q = q_ref[i]
        ss = [
            jax.lax.dot_general(
                q, kr[i], (((1,), (1,)), ((), ())),
                preferred_element_type=jnp.float32,
            ) * scale
            for kr in k_refs
        ]
        m = functools.reduce(jnp.maximum, [jnp.max(s) for s in ss])
        ps = [jnp.exp(s - m) for s in ss]
        denom = functools.reduce(jnp.add, [jnp.sum(p) for p in ps])
        inv = 1.0 / denom
        ctx = jnp.zeros((1, DIM), jnp.float32)
        for c, (p, vr) in enumerate(zip(ps, v_refs)):
            a = p * inv
            ctx = ctx + jnp.dot(a, vr[i], preferred_element_type=jnp.float32)
            attn_ref[i, :, c * CHUNK:(c + 1) * CHUNK] = a
        ctx_ref[i] = ctx


@jax.jit
def kernel(query, key, value):
    batch, q_len, dim = query.shape
    kv_len = key.shape[1]
    chunk = kv_len // NSPLIT
    grid = (batch // BB,)
    out_ctx = jax.ShapeDtypeStruct((batch, q_len, dim), jnp.float32)
    out_attn = jax.ShapeDtypeStruct((batch, q_len, kv_len), jnp.float32)

    def _kv_spec(c):
        return pl.BlockSpec((BB, chunk, dim), lambda b, c=c: (b, c, 0))

    ctx, attn = pl.pallas_call(
        _attn_kernel,
        grid=grid,
        in_specs=(
            [pl.BlockSpec((BB, q_len, dim), lambda b: (b, 0, 0))]
            + [_kv_spec(c) for c in range(NSPLIT)]
            + [_kv_spec(c) for c in range(NSPLIT)]
        ),
        out_specs=[
            pl.BlockSpec((BB, q_len, dim), lambda b: (b, 0, 0)),
            pl.BlockSpec((BB, q_len, kv_len), lambda b: (b, 0, 0)),
        ],
        out_shape=[out_ctx, out_attn],
        compiler_params=pltpu.CompilerParams(
            dimension_semantics=("parallel",),
        ),
    )(query, *([key] * NSPLIT), *([value] * NSPLIT))
    return (ctx, attn)


# speedup vs baseline: 1.0011x; 1.0011x over previous
import functools
import math

import jax
import jax.numpy as jnp
from jax.experimental import pallas as pl
from jax.experimental.pallas import tpu as pltpu

DIM = 128
KV_LEN = 8192
BATCH = 128
NK = 4
NC = 2 * NK
CH = KV_LEN // NK
NBUF = 2 * NC
E = NC


def _issue(b, j, key_hbm, value_hbm, kv_buf, sems):
    slot = jax.lax.rem(b, 2) * NC + j
    if j < NK:
        src = key_hbm.at[b, pl.ds(j * CH, CH), :]
    else:
        src = value_hbm.at[b, pl.ds((j - NK) * CH, CH), :]
    pltpu.make_async_copy(src, kv_buf.at[slot], sems.at[slot]).start()


def _wait(b, j, key_hbm, value_hbm, kv_buf, sems):
    slot = jax.lax.rem(b, 2) * NC + j
    if j < NK:
        src = key_hbm.at[b, pl.ds(j * CH, CH), :]
    else:
        src = value_hbm.at[b, pl.ds((j - NK) * CH, CH), :]
    pltpu.make_async_copy(src, kv_buf.at[slot], sems.at[slot]).wait()
    return slot


def _attn_kernel(q_ref, key_hbm, value_hbm, ctx_ref, attn_ref, kv_buf, sems):
    b = pl.program_id(0)
    scale = 1.0 / math.sqrt(DIM)

    @pl.when(b == 0)
    def _():
        for j in range(NC):
            _issue(0, j, key_hbm, value_hbm, kv_buf, sems)

    q = q_ref[0]

    ss = []
    for j in range(NK):
        slot = _wait(b, j, key_hbm, value_hbm, kv_buf, sems)

        @pl.when(b < BATCH - 1)
        def _(j=j):
            _issue(b + 1, j, key_hbm, value_hbm, kv_buf, sems)

        k = kv_buf[slot]
        ss.append(
            jax.lax.dot_general(
                q, k, (((1,), (1,)), ((), ())),
                preferred_element_type=jnp.float32,
            ) * scale
        )

    m = functools.reduce(jnp.maximum, [jnp.max(s) for s in ss])
    ps = [jnp.exp(s - m) for s in ss]
    denom = functools.reduce(jnp.add, [jnp.sum(p) for p in ps])
    inv = 1.0 / denom

    ctx = jnp.zeros((1, DIM), jnp.float32)
    for c in range(NK):
        j = NK + c
        slot = _wait(b, j, key_hbm, value_hbm, kv_buf, sems)

        @pl.when(b < BATCH - 1)
        def _(j=j):
            _issue(b + 1, j, key_hbm, value_hbm, kv_buf, sems)

        a = ps[c] * inv
        v = kv_buf[slot]
        ctx = ctx + jnp.dot(a, v, preferred_element_type=jnp.float32)
        attn_ref[0, :, c * CH:(c + 1) * CH] = a

    ctx_ref[0] = ctx


@jax.jit
def kernel(query, key, value):
    batch, q_len, dim = query.shape
    kv_len = key.shape[1]
    out_ctx = jax.ShapeDtypeStruct((batch, q_len, dim), jnp.float32)
    out_attn = jax.ShapeDtypeStruct((batch, q_len, kv_len), jnp.float32)
    ctx, attn = pl.pallas_call(
        _attn_kernel,
        grid=(batch,),
        in_specs=[
            pl.BlockSpec((1, q_len, dim), lambda b: (b, 0, 0)),
            pl.BlockSpec(memory_space=pltpu.MemorySpace.HBM),
            pl.BlockSpec(memory_space=pltpu.MemorySpace.HBM),
        ],
        out_specs=[
            pl.BlockSpec((1, q_len, dim), lambda b: (b, 0, 0)),
            pl.BlockSpec((1, q_len, kv_len), lambda b: (b, 0, 0)),
        ],
        out_shape=[out_ctx, out_attn],
        scratch_shapes=[
            pltpu.VMEM((NBUF, CH, DIM), jnp.float32),
            pltpu.SemaphoreType.DMA((NBUF,)),
        ],
        compiler_params=pltpu.CompilerParams(
            dimension_semantics=("arbitrary",),
        ),
    )(query, key, value)
    return (ctx, attn)
